# baseline (speedup 1.0000x reference)
import functools

import jax
import jax.numpy as jnp
from jax import lax
from jax.experimental import pallas as pl
from jax.experimental.pallas import tpu as pltpu
from jax.experimental.pallas import tpu_sc as plsc

H = 16
N_NODES = 10000
NUM_ATOMS = 28

_NC = 2
_NS = 16
_NW = _NC * _NS
_CHUNK = 128
_E_PAD = 163840
_CPW = _E_PAD // (_NW * _CHUNK)
_BPW = _E_PAD // _NW
_N_PAD = 10240
_NPW = _N_PAD // _NW
_NPS = _N_PAD // _NS

_SC_PARAMS = pltpu.CompilerParams(use_tc_tiling_on_sc=False,
                                 needs_layout_passes=False)


def _mesh():
    return plsc.VectorSubcoreMesh(
        core_axis_name="c", subcore_axis_name="s", num_cores=_NC)


def _sc_make_table(hT):
    @functools.partial(
        pl.kernel, mesh=_mesh(), compiler_params=_SC_PARAMS,
        out_type=jax.ShapeDtypeStruct((_N_PAD, H), jnp.float32),
        scratch_types=[
            pltpu.VMEM((H, _NPW), jnp.float32),
            pltpu.VMEM((_NPW, H), jnp.float32),
        ],
    )
    def k(hT_hbm, out_hbm, colbuf, rowbuf):
        wid = lax.axis_index("s") * _NC + lax.axis_index("c")
        base = wid * _NPW
        pltpu.sync_copy(hT_hbm.at[:, pl.ds(base, _NPW)], colbuf)
        lanes = lax.iota(jnp.int32, 16)
        def body(b, _):
            for rr in range(16):
                r = b * 16 + rr
                vals = plsc.load_gather(
                    colbuf, [lanes, jnp.zeros((16,), jnp.int32) + r])
                rowbuf[r, :] = vals
            return 0
        lax.fori_loop(0, _NPW // 16, body, 0)
        pltpu.sync_copy(rowbuf, out_hbm.at[pl.ds(base, _NPW)])

    return k(hT)


def _sc_gather_T(table, idx2):
    @functools.partial(
        pl.kernel, mesh=_mesh(), compiler_params=_SC_PARAMS,
        out_type=jax.ShapeDtypeStruct((H, _E_PAD), jnp.float32),
        scratch_types=[
            pltpu.VMEM((_CPW, _CHUNK), jnp.int32),
            [pltpu.VMEM((_CHUNK, H), jnp.float32) for _ in range(8)],
            [pltpu.VMEM((H, _CHUNK), jnp.float32) for _ in range(4)],
            [pltpu.SemaphoreType.DMA for _ in range(8)],
            [pltpu.SemaphoreType.DMA for _ in range(4)],
        ],
    )
    def k(table_hbm, idx_hbm, out_hbm, idx_v, rows, cols, semg, semo):
        wid = lax.axis_index("s") * _NC + lax.axis_index("c")
        base = wid * _BPW
        pltpu.sync_copy(idx_hbm.at[pl.ds(wid * _CPW, _CPW)], idx_v)
        lanes = lax.iota(jnp.int32, 16)
        for p in range(8):
            pltpu.async_copy(table_hbm.at[idx_v.at[p]], rows[p], semg[p])

        def body(j8, _):
            for p in range(8):
                j = j8 * 8 + p
                pc = p % 4
                pltpu.make_async_copy(
                    table_hbm.at[idx_v.at[j]], rows[p], semg[p]).wait()
                if p >= 4:
                    pltpu.make_async_copy(
                        cols[pc],
                        out_hbm.at[:, pl.ds(base + (j - 4) * _CHUNK, _CHUNK)],
                        semo[pc]).wait()
                else:
                    @pl.when(j8 > 0)
                    def _():
                        pltpu.make_async_copy(
                            cols[pc],
                            out_hbm.at[:,
                                       pl.ds(base + (j - 4) * _CHUNK, _CHUNK)],
                            semo[pc]).wait()
                for g in range(_CHUNK // 16):
                    rws = g * 16 + lanes
                    for i in range(H):
                        vals = plsc.load_gather(
                            rows[p], [rws, jnp.full((16,), i, jnp.int32)])
                        cols[pc][i, pl.ds(g * 16, 16)] = vals
                pltpu.async_copy(
                    cols[pc],
                    out_hbm.at[:, pl.ds(base + j * _CHUNK, _CHUNK)], semo[pc])
                @pl.when(j8 < _CPW // 8 - 1)
                def _():
                    pltpu.async_copy(
                        table_hbm.at[idx_v.at[j + 8]], rows[p], semg[p])
            return 0
        lax.fori_loop(0, _CPW // 8, body, 0)
        for j in range(_CPW - 4, _CPW):
            pltpu.make_async_copy(
                cols[j % 4],
                out_hbm.at[:, pl.ds(base + j * _CHUNK, _CHUNK)],
                semo[j % 4]).wait()

    return k(table, idx2)


def _sc_scatter_T(msgT, dst2, zeros):
    @functools.partial(
        pl.kernel, mesh=_mesh(), compiler_params=_SC_PARAMS,
        out_type=jax.ShapeDtypeStruct((_NC, H, _N_PAD), jnp.float32),
        scratch_types=[
            pltpu.VMEM((_CPW, _CHUNK), jnp.int32),
            [pltpu.VMEM((H, _CHUNK), jnp.float32) for _ in range(4)],
            [pltpu.VMEM((_CHUNK, H), jnp.float32) for _ in range(2)],
            pltpu.VMEM((H, _NPS), jnp.float32),
            pltpu.VMEM((16, H), jnp.float32),
            pltpu.VMEM_SHARED((_N_PAD, H), jnp.float32),
            [pltpu.SemaphoreType.DMA for _ in range(4)],
            [pltpu.SemaphoreType.DMA for _ in range(2)],
        ],
    )
    def k(msg_hbm, dst_hbm, zeros_hbm, out_hbm, idx_v, cols, rows,
          colbuf, tmp_v, aggr_sh, semr, semw):
        c = lax.axis_index("c")
        s = lax.axis_index("s")
        wid = s * _NC + c
        base = wid * _BPW
        pltpu.sync_copy(zeros_hbm.at[pl.ds(s * _NPS, _NPS)],
                        aggr_sh.at[pl.ds(s * _NPS, _NPS)])
        pltpu.sync_copy(dst_hbm.at[pl.ds(wid * _CPW, _CPW)], idx_v)
        plsc.subcore_barrier()
        lanes = lax.iota(jnp.int32, 16)
        for p in range(4):
            pltpu.async_copy(
                msg_hbm.at[:, pl.ds(base + p * _CHUNK, _CHUNK)],
                cols[p], semr[p])

        def body(j4, _):
            for p in range(4):
                j = j4 * 4 + p
                pr = p % 2
                pltpu.make_async_copy(
                    msg_hbm.at[:, pl.ds(base + j * _CHUNK, _CHUNK)],
                    cols[p], semr[p]).wait()
                if p >= 2:
                    pltpu.make_async_copy(
                        rows[pr], aggr_sh.at[idx_v.at[j - 2]],
                        semw[pr]).wait()
                else:
                    @pl.when(j4 > 0)
                    def _():
                        pltpu.make_async_copy(
                            rows[pr], aggr_sh.at[idx_v.at[j - 2]],
                            semw[pr]).wait()
                for e in range(_CHUNK):
                    vals = plsc.load_gather(
                        cols[p], [lanes, jnp.zeros((16,), jnp.int32) + e])
                    rows[pr][e, :] = vals
                pltpu.async_copy(rows[pr], aggr_sh.at[idx_v.at[j]],
                                 semw[pr], add=True)
                @pl.when(j4 < _CPW // 4 - 1)
                def _():
                    pltpu.async_copy(
                        msg_hbm.at[:, pl.ds(base + (j + 4) * _CHUNK, _CHUNK)],
                        cols[p], semr[p])
            return 0
        lax.fori_loop(0, _CPW // 4, body, 0)
        for j in (_CPW - 2, _CPW - 1):
            pltpu.make_async_copy(
                rows[j % 2], aggr_sh.at[idx_v.at[j]], semw[j % 2]).wait()
        plsc.subcore_barrier()
        def tbody(b, _):
            pltpu.sync_copy(aggr_sh.at[pl.ds(s * _NPS + b * 16, 16)], tmp_v)
            for i in range(H):
                vals = plsc.load_gather(
                    tmp_v, [lanes, jnp.full((16,), i, jnp.int32)])
                colbuf[i, pl.ds(b * 16, 16)] = vals
            return 0
        lax.fori_loop(0, _NPS // 16, tbody, 0)
        pltpu.sync_copy(colbuf,
                        out_hbm.at[c].at[:, pl.ds(s * _NPS, _NPS)])

    return k(msgT, dst2, zeros)


def _edge_mlp_T_body(eaT_ref, w1T_ref, b1_ref, w2T_ref, b2_ref, out_ref, *,
                     n_edges):
    BET = eaT_ref.shape[1]
    hmid = jnp.maximum(w1T_ref[...] @ eaT_ref[...] + b1_ref[...], 0.0)
    val = w2T_ref[...] @ hmid + b2_ref[...]
    col = pl.program_id(0) * BET + lax.broadcasted_iota(
        jnp.int32, (1, BET), 1)
    out_ref[...] = jnp.where(col < n_edges, val, 0.0).astype(jnp.bfloat16)


def _edge_mlp_T(eaT_p, n_edges, eW1, eb1, eW2, eb2):
    EP = eaT_p.shape[1]
    BET = 2048
    return pl.pallas_call(
        functools.partial(_edge_mlp_T_body, n_edges=n_edges),
        grid=(EP // BET,),
        in_specs=[
            pl.BlockSpec((4, BET), lambda i: (0, i)),
            pl.BlockSpec((128, 4), lambda i: (0, 0)),
            pl.BlockSpec((128, 1), lambda i: (0, 0)),
            pl.BlockSpec((256, 128), lambda i: (0, 0)),
            pl.BlockSpec((256, 1), lambda i: (0, 0)),
        ],
        out_specs=pl.BlockSpec((256, BET), lambda i: (0, i)),
        out_shape=jax.ShapeDtypeStruct((256, EP), jnp.bfloat16),
    )(eaT_p, eW1.T, eb1[:, None], eW2.T, eb2[:, None])


def _msg_T_body(xgT_ref, ewT_ref, out_ref):
    ew = ewT_ref[...].astype(jnp.float32)
    acc = xgT_ref[0:1, :] * ew[0:H, :]
    for i in range(1, H):
        acc += xgT_ref[i:i + 1, :] * ew[i * H:(i + 1) * H, :]
    out_ref[...] = acc


def _msg_T(xgT, ewT):
    EP = xgT.shape[1]
    BET = 4096
    return pl.pallas_call(
        _msg_T_body,
        grid=(EP // BET,),
        in_specs=[
            pl.BlockSpec((H, BET), lambda i: (0, i)),
            pl.BlockSpec((256, BET), lambda i: (0, i)),
        ],
        out_specs=pl.BlockSpec((H, BET), lambda i: (0, i)),
        out_shape=jax.ShapeDtypeStruct((H, EP), jnp.float32),
    )(xgT, ewT)


def _init_T_body(xT_ref, embT_ref, extpT_ref, out_ref):
    atom = lax.broadcasted_iota(jnp.int32, (NUM_ATOMS, xT_ref.shape[1]), 0)
    oh = jnp.where(atom == xT_ref[...], 1.0, 0.0)
    out_ref[...] = jnp.maximum(embT_ref[...] @ oh, 0.0) + extpT_ref[...]


def _init_T(xT_p, extpT, emb):
    embT_p = jnp.pad(emb.T, ((0, H - emb.shape[1]), (0, 0)))
    BN = 2048
    return pl.pallas_call(
        _init_T_body,
        grid=(_N_PAD // BN,),
        in_specs=[
            pl.BlockSpec((1, BN), lambda i: (0, i)),
            pl.BlockSpec((H, NUM_ATOMS), lambda i: (0, 0)),
            pl.BlockSpec((H, BN), lambda i: (0, i)),
        ],
        out_specs=pl.BlockSpec((H, BN), lambda i: (0, i)),
        out_shape=jax.ShapeDtypeStruct((H, _N_PAD), jnp.float32),
    )(xT_p, embT_p, extpT)


def _gru_T_body(p_ref, h_ref, rootT_ref, cb_ref, wihT_ref, whhT_ref, bih_ref,
                bhh_ref, out_ref):
    h = h_ref[...]
    aggr = p_ref[0] + p_ref[1]
    m = jnp.maximum(aggr + rootT_ref[...] @ h + cb_ref[...], 0.0)
    gi = wihT_ref[...] @ m + bih_ref[...]
    gh = whhT_ref[...] @ h + bhh_ref[...]
    r = jax.nn.sigmoid(gi[0:H, :] + gh[0:H, :])
    z = jax.nn.sigmoid(gi[H:2 * H, :] + gh[H:2 * H, :])
    n_ = jnp.tanh(gi[2 * H:3 * H, :] + r * gh[2 * H:3 * H, :])
    out_ref[...] = (1.0 - z) * n_ + z * h


def _gru_T(partialsT, hT, root, conv_b, Wih, Whh, bih, bhh):
    BN = 2048
    return pl.pallas_call(
        _gru_T_body,
        grid=(_N_PAD // BN,),
        in_specs=[
            pl.BlockSpec((2, H, BN), lambda i: (0, 0, i)),
            pl.BlockSpec((H, BN), lambda i: (0, i)),
            pl.BlockSpec((H, H), lambda i: (0, 0)),
            pl.BlockSpec((H, 1), lambda i: (0, 0)),
            pl.BlockSpec((3 * H, H), lambda i: (0, 0)),
            pl.BlockSpec((3 * H, H), lambda i: (0, 0)),
            pl.BlockSpec((3 * H, 1), lambda i: (0, 0)),
            pl.BlockSpec((3 * H, 1), lambda i: (0, 0)),
        ],
        out_specs=pl.BlockSpec((H, BN), lambda i: (0, i)),
        out_shape=jax.ShapeDtypeStruct((H, _N_PAD), jnp.float32),
    )(partialsT, hT, root.T, conv_b[:, None], Wih.T, Whh.T, bih[:, None],
      bhh[:, None])


def _final_T_body(h_ref, xT_ref, extT_ref, wo1T_ref, bo1_ref, wo2T_ref,
                  bo2_ref, wc1T_ref, bc1_ref, wc2T_ref, bc2_ref, fg_ref,
                  pred_ref):
    BN = h_ref.shape[1]
    t = jnp.maximum(wo1T_ref[...] @ h_ref[...] + bo1_ref[...], 0.0)
    t = wo2T_ref[...] @ t + bo2_ref[...]
    atom = lax.broadcasted_iota(jnp.int32, (NUM_ATOMS, BN), 0)
    oh = jnp.where(atom == xT_ref[...], 1.0, 0.0)
    feat = jnp.concatenate([t, oh, extT_ref[...]], axis=0)
    nrm = jnp.sqrt(jnp.sum(feat * feat, axis=0, keepdims=True))
    fg = feat / jnp.maximum(nrm, 1e-12)
    fg_ref[...] = fg
    p = jnp.maximum(wc1T_ref[...] @ fg + bc1_ref[...], 0.0)
    pred_ref[...] = wc2T_ref[...] @ p + bc2_ref[...]


def _final_T(hT, xT_p, extT_p, Wo1, bo1, Wo2, bo2, Wc1, bc1, Wc2, bc2):
    FEAT = 64 + NUM_ATOMS + 3
    BN = 2048
    return pl.pallas_call(
        _final_T_body,
        grid=(_N_PAD // BN,),
        in_specs=[
            pl.BlockSpec((H, BN), lambda i: (0, i)),
            pl.BlockSpec((1, BN), lambda i: (0, i)),
            pl.BlockSpec((3, BN), lambda i: (0, i)),
            pl.BlockSpec((H, H), lambda i: (0, 0)),
            pl.BlockSpec((H, 1), lambda i: (0, 0)),
            pl.BlockSpec((64, H), lambda i: (0, 0)),
            pl.BlockSpec((64, 1), lambda i: (0, 0)),
            pl.BlockSpec((256, FEAT), lambda i: (0, 0)),
            pl.BlockSpec((256, 1), lambda i: (0, 0)),
            pl.BlockSpec((18, 256), lambda i: (0, 0)),
            pl.BlockSpec((18, 1), lambda i: (0, 0)),
        ],
        out_specs=[
            pl.BlockSpec((FEAT, BN), lambda i: (0, i)),
            pl.BlockSpec((18, BN), lambda i: (0, i)),
        ],
        out_shape=[
            jax.ShapeDtypeStruct((FEAT, _N_PAD), jnp.float32),
            jax.ShapeDtypeStruct((18, _N_PAD), jnp.float32),
        ],
    )(hT, xT_p, extT_p, Wo1.T, bo1[:, None], Wo2.T, bo2[:, None],
      Wc1.T, bc1[:, None], Wc2.T, bc2[:, None])


def kernel(x, edge_index, edge_attr, extended_feat, emb, eW1, eb1, eW2, eb2,
           root, conv_b, Wih, Whh, bih, bhh, Wo1, bo1, Wo2, bo2, Wc1, bc1,
           Wc2, bc2):
    E = edge_index.shape[1]
    src2 = jnp.pad(edge_index[0], (0, _E_PAD - E)).reshape(-1, _CHUNK)
    dst2 = jnp.pad(edge_index[1], (0, _E_PAD - E)).reshape(-1, _CHUNK)
    eaT_p = jnp.pad(edge_attr.T, ((0, 0), (0, _E_PAD - E)))
    xT_p = jnp.pad(x.T, ((0, 0), (0, _N_PAD - N_NODES)))
    extT_p = jnp.pad(extended_feat.T, ((0, 0), (0, _N_PAD - N_NODES)))
    extpT = jnp.pad(extended_feat.T,
                    ((H - 3, 0), (0, _N_PAD - N_NODES)))
    zeros = jnp.zeros((_N_PAD, H), jnp.float32)

    ewT = _edge_mlp_T(eaT_p, E, eW1, eb1, eW2, eb2)
    hT = _init_T(xT_p, extpT, emb)
    for _ in range(6):
        table = _sc_make_table(hT)
        xgT = _sc_gather_T(table, src2)
        msgT = _msg_T(xgT, ewT)
        partialsT = _sc_scatter_T(msgT, dst2, zeros)
        hT = _gru_T(partialsT, hT, root, conv_b, Wih, Whh, bih, bhh)
    fgT, predT = _final_T(hT, xT_p, extT_p, Wo1, bo1, Wo2, bo2, Wc1, bc1,
                          Wc2, bc2)
    fg = fgT[:, :N_NODES].T
    pred = predT[:, :N_NODES].T
    return (fg, pred)


# speedup vs baseline: 1.0191x; 1.0191x over previous
import functools

import jax
import jax.numpy as jnp
from jax import lax
from jax.experimental import pallas as pl
from jax.experimental.pallas import tpu as pltpu
from jax.experimental.pallas import tpu_sc as plsc

H = 16
N_NODES = 10000
NUM_ATOMS = 28

_NC = 2
_NS = 16
_NW = _NC * _NS
_CHUNK = 128
_E_PAD = 163840
_CPW = _E_PAD // (_NW * _CHUNK)
_BPW = _E_PAD // _NW
_N_PAD = 10240
_NPW = _N_PAD // _NW
_NPS = _N_PAD // _NS

_SC_PARAMS = pltpu.CompilerParams(use_tc_tiling_on_sc=False,
                                 needs_layout_passes=False)


def _mesh():
    return plsc.VectorSubcoreMesh(
        core_axis_name="c", subcore_axis_name="s", num_cores=_NC)


def _sc_make_table(hT):
    @functools.partial(
        pl.kernel, mesh=_mesh(), compiler_params=_SC_PARAMS,
        out_type=jax.ShapeDtypeStruct((_N_PAD, H), jnp.float32),
        scratch_types=[
            pltpu.VMEM((H, _NPW), jnp.float32),
            pltpu.VMEM((_NPW, H), jnp.float32),
        ],
    )
    def k(hT_hbm, out_hbm, colbuf, rowbuf):
        wid = lax.axis_index("s") * _NC + lax.axis_index("c")
        base = wid * _NPW
        pltpu.sync_copy(hT_hbm.at[:, pl.ds(base, _NPW)], colbuf)
        lanes = lax.iota(jnp.int32, 16)
        def body(b, _):
            for rr in range(16):
                r = b * 16 + rr
                vals = plsc.load_gather(
                    colbuf, [lanes, jnp.zeros((16,), jnp.int32) + r])
                rowbuf[r, :] = vals
            return 0
        lax.fori_loop(0, _NPW // 16, body, 0)
        pltpu.sync_copy(rowbuf, out_hbm.at[pl.ds(base, _NPW)])

    return k(hT)


def _sc_gather_T(hT, idx2):
    @functools.partial(
        pl.kernel, mesh=_mesh(), compiler_params=_SC_PARAMS,
        out_type=jax.ShapeDtypeStruct((H, _E_PAD), jnp.float32),
        scratch_types=[
            pltpu.VMEM((_CPW, _CHUNK), jnp.int32),
            pltpu.VMEM((H, _NPS), jnp.float32),
            pltpu.VMEM((_NPS, H), jnp.float32),
            plsc.MemoryRef if False else pltpu.VMEM_SHARED((_N_PAD, H), jnp.float32),
            [pltpu.VMEM((_CHUNK, H), jnp.float32) for _ in range(4)],
            [pltpu.VMEM((H, _CHUNK), jnp.float32) for _ in range(2)],
            [pltpu.SemaphoreType.DMA for _ in range(4)],
            [pltpu.SemaphoreType.DMA for _ in range(2)],
        ],
    )
    def k(hT_hbm, idx_hbm, out_hbm, idx_v, tblbuf, rowtmp, table_sh, rows,
          cols, semg, semo):
        c = lax.axis_index("c")
        s = lax.axis_index("s")
        wid = s * _NC + c
        base = wid * _BPW
        pltpu.sync_copy(idx_hbm.at[pl.ds(wid * _CPW, _CPW)], idx_v)
        lanes = lax.iota(jnp.int32, 16)
        pltpu.sync_copy(hT_hbm.at[:, pl.ds(s * _NPS, _NPS)], tblbuf)
        def tbody(b, _):
            for rr in range(16):
                r = b * 16 + rr
                vals = plsc.load_gather(
                    tblbuf, [lanes, jnp.zeros((16,), jnp.int32) + r])
                rowtmp[r, :] = vals
            return 0
        lax.fori_loop(0, _NPS // 16, tbody, 0)
        pltpu.sync_copy(rowtmp, table_sh.at[pl.ds(s * _NPS, _NPS)])
        plsc.subcore_barrier()

        for p in range(4):
            pltpu.async_copy(table_sh.at[idx_v.at[p]], rows[p], semg[p])

        def body(j4, _):
            for p in range(4):
                j = j4 * 4 + p
                pc = p % 2
                pltpu.make_async_copy(
                    table_sh.at[idx_v.at[j]], rows[p], semg[p]).wait()
                if p >= 2:
                    pltpu.make_async_copy(
                        cols[pc],
                        out_hbm.at[:, pl.ds(base + (j - 2) * _CHUNK, _CHUNK)],
                        semo[pc]).wait()
                else:
                    @pl.when(j4 > 0)
                    def _():
                        pltpu.make_async_copy(
                            cols[pc],
                            out_hbm.at[:,
                                       pl.ds(base + (j - 2) * _CHUNK, _CHUNK)],
                            semo[pc]).wait()
                for g in range(_CHUNK // 16):
                    rws = g * 16 + lanes
                    for i in range(H):
                        vals = plsc.load_gather(
                            rows[p], [rws, jnp.full((16,), i, jnp.int32)])
                        cols[pc][i, pl.ds(g * 16, 16)] = vals
                pltpu.async_copy(
                    cols[pc],
                    out_hbm.at[:, pl.ds(base + j * _CHUNK, _CHUNK)], semo[pc])
                @pl.when(j4 < _CPW // 4 - 1)
                def _():
                    pltpu.async_copy(
                        table_sh.at[idx_v.at[j + 4]], rows[p], semg[p])
            return 0
        lax.fori_loop(0, _CPW // 4, body, 0)
        for j in (_CPW - 2, _CPW - 1):
            pltpu.make_async_copy(
                cols[j % 2],
                out_hbm.at[:, pl.ds(base + j * _CHUNK, _CHUNK)],
                semo[j % 2]).wait()

    return k(hT, idx2)


def _sc_scatter_T(msgT, dst2, zeros):
    @functools.partial(
        pl.kernel, mesh=_mesh(), compiler_params=_SC_PARAMS,
        out_type=jax.ShapeDtypeStruct((_NC, H, _N_PAD), jnp.float32),
        scratch_types=[
            pltpu.VMEM((_CPW, _CHUNK), jnp.int32),
            [pltpu.VMEM((H, _CHUNK), jnp.float32) for _ in range(4)],
            [pltpu.VMEM((_CHUNK, H), jnp.float32) for _ in range(2)],
            pltpu.VMEM((H, _NPS), jnp.float32),
            pltpu.VMEM((16, H), jnp.float32),
            pltpu.VMEM_SHARED((_N_PAD, H), jnp.float32),
            [pltpu.SemaphoreType.DMA for _ in range(4)],
            [pltpu.SemaphoreType.DMA for _ in range(2)],
        ],
    )
    def k(msg_hbm, dst_hbm, zeros_hbm, out_hbm, idx_v, cols, rows,
          colbuf, tmp_v, aggr_sh, semr, semw):
        c = lax.axis_index("c")
        s = lax.axis_index("s")
        wid = s * _NC + c
        base = wid * _BPW
        pltpu.sync_copy(zeros_hbm.at[pl.ds(s * _NPS, _NPS)],
                        aggr_sh.at[pl.ds(s * _NPS, _NPS)])
        pltpu.sync_copy(dst_hbm.at[pl.ds(wid * _CPW, _CPW)], idx_v)
        plsc.subcore_barrier()
        lanes = lax.iota(jnp.int32, 16)
        for p in range(4):
            pltpu.async_copy(
                msg_hbm.at[:, pl.ds(base + p * _CHUNK, _CHUNK)],
                cols[p], semr[p])

        def body(j4, _):
            for p in range(4):
                j = j4 * 4 + p
                pr = p % 2
                pltpu.make_async_copy(
                    msg_hbm.at[:, pl.ds(base + j * _CHUNK, _CHUNK)],
                    cols[p], semr[p]).wait()
                if p >= 2:
                    pltpu.make_async_copy(
                        rows[pr], aggr_sh.at[idx_v.at[j - 2]],
                        semw[pr]).wait()
                else:
                    @pl.when(j4 > 0)
                    def _():
                        pltpu.make_async_copy(
                            rows[pr], aggr_sh.at[idx_v.at[j - 2]],
                            semw[pr]).wait()
                for e in range(_CHUNK):
                    vals = plsc.load_gather(
                        cols[p], [lanes, jnp.zeros((16,), jnp.int32) + e])
                    rows[pr][e, :] = vals
                pltpu.async_copy(rows[pr], aggr_sh.at[idx_v.at[j]],
                                 semw[pr], add=True)
                @pl.when(j4 < _CPW // 4 - 1)
                def _():
                    pltpu.async_copy(
                        msg_hbm.at[:, pl.ds(base + (j + 4) * _CHUNK, _CHUNK)],
                        cols[p], semr[p])
            return 0
        lax.fori_loop(0, _CPW // 4, body, 0)
        for j in (_CPW - 2, _CPW - 1):
            pltpu.make_async_copy(
                rows[j % 2], aggr_sh.at[idx_v.at[j]], semw[j % 2]).wait()
        plsc.subcore_barrier()
        def tbody(b, _):
            pltpu.sync_copy(aggr_sh.at[pl.ds(s * _NPS + b * 16, 16)], tmp_v)
            for i in range(H):
                vals = plsc.load_gather(
                    tmp_v, [lanes, jnp.full((16,), i, jnp.int32)])
                colbuf[i, pl.ds(b * 16, 16)] = vals
            return 0
        lax.fori_loop(0, _NPS // 16, tbody, 0)
        pltpu.sync_copy(colbuf,
                        out_hbm.at[c].at[:, pl.ds(s * _NPS, _NPS)])

    return k(msgT, dst2, zeros)


def _edge_mlp_T_body(eaT_ref, w1T_ref, b1_ref, w2T_ref, b2_ref, out_ref, *,
                     n_edges):
    BET = eaT_ref.shape[1]
    hmid = jnp.maximum(w1T_ref[...] @ eaT_ref[...] + b1_ref[...], 0.0)
    val = w2T_ref[...] @ hmid + b2_ref[...]
    col = pl.program_id(0) * BET + lax.broadcasted_iota(
        jnp.int32, (1, BET), 1)
    out_ref[...] = jnp.where(col < n_edges, val, 0.0).astype(jnp.bfloat16)


def _edge_mlp_T(eaT_p, n_edges, eW1, eb1, eW2, eb2):
    EP = eaT_p.shape[1]
    BET = 2048
    return pl.pallas_call(
        functools.partial(_edge_mlp_T_body, n_edges=n_edges),
        grid=(EP // BET,),
        in_specs=[
            pl.BlockSpec((4, BET), lambda i: (0, i)),
            pl.BlockSpec((128, 4), lambda i: (0, 0)),
            pl.BlockSpec((128, 1), lambda i: (0, 0)),
            pl.BlockSpec((256, 128), lambda i: (0, 0)),
            pl.BlockSpec((256, 1), lambda i: (0, 0)),
        ],
        out_specs=pl.BlockSpec((256, BET), lambda i: (0, i)),
        out_shape=jax.ShapeDtypeStruct((256, EP), jnp.bfloat16),
    )(eaT_p, eW1.T, eb1[:, None], eW2.T, eb2[:, None])


def _msg_T_body(xgT_ref, ewT_ref, out_ref):
    ew = ewT_ref[...].astype(jnp.float32)
    acc = xgT_ref[0:1, :] * ew[0:H, :]
    for i in range(1, H):
        acc += xgT_ref[i:i + 1, :] * ew[i * H:(i + 1) * H, :]
    out_ref[...] = acc


def _msg_T(xgT, ewT):
    EP = xgT.shape[1]
    BET = 4096
    return pl.pallas_call(
        _msg_T_body,
        grid=(EP // BET,),
        in_specs=[
            pl.BlockSpec((H, BET), lambda i: (0, i)),
            pl.BlockSpec((256, BET), lambda i: (0, i)),
        ],
        out_specs=pl.BlockSpec((H, BET), lambda i: (0, i)),
        out_shape=jax.ShapeDtypeStruct((H, EP), jnp.float32),
    )(xgT, ewT)


def _init_T_body(xT_ref, embT_ref, extpT_ref, out_ref):
    atom = lax.broadcasted_iota(jnp.int32, (NUM_ATOMS, xT_ref.shape[1]), 0)
    oh = jnp.where(atom == xT_ref[...], 1.0, 0.0)
    out_ref[...] = jnp.maximum(embT_ref[...] @ oh, 0.0) + extpT_ref[...]


def _init_T(xT_p, extpT, emb):
    embT_p = jnp.pad(emb.T, ((0, H - emb.shape[1]), (0, 0)))
    BN = 2048
    return pl.pallas_call(
        _init_T_body,
        grid=(_N_PAD // BN,),
        in_specs=[
            pl.BlockSpec((1, BN), lambda i: (0, i)),
            pl.BlockSpec((H, NUM_ATOMS), lambda i: (0, 0)),
            pl.BlockSpec((H, BN), lambda i: (0, i)),
        ],
        out_specs=pl.BlockSpec((H, BN), lambda i: (0, i)),
        out_shape=jax.ShapeDtypeStruct((H, _N_PAD), jnp.float32),
    )(xT_p, embT_p, extpT)


def _gru_T_body(p_ref, h_ref, rootT_ref, cb_ref, wihT_ref, whhT_ref, bih_ref,
                bhh_ref, out_ref):
    h = h_ref[...]
    aggr = p_ref[0] + p_ref[1]
    m = jnp.maximum(aggr + rootT_ref[...] @ h + cb_ref[...], 0.0)
    gi = wihT_ref[...] @ m + bih_ref[...]
    gh = whhT_ref[...] @ h + bhh_ref[...]
    r = jax.nn.sigmoid(gi[0:H, :] + gh[0:H, :])
    z = jax.nn.sigmoid(gi[H:2 * H, :] + gh[H:2 * H, :])
    n_ = jnp.tanh(gi[2 * H:3 * H, :] + r * gh[2 * H:3 * H, :])
    out_ref[...] = (1.0 - z) * n_ + z * h


def _gru_T(partialsT, hT, root, conv_b, Wih, Whh, bih, bhh):
    BN = 2048
    return pl.pallas_call(
        _gru_T_body,
        grid=(_N_PAD // BN,),
        in_specs=[
            pl.BlockSpec((2, H, BN), lambda i: (0, 0, i)),
            pl.BlockSpec((H, BN), lambda i: (0, i)),
            pl.BlockSpec((H, H), lambda i: (0, 0)),
            pl.BlockSpec((H, 1), lambda i: (0, 0)),
            pl.BlockSpec((3 * H, H), lambda i: (0, 0)),
            pl.BlockSpec((3 * H, H), lambda i: (0, 0)),
            pl.BlockSpec((3 * H, 1), lambda i: (0, 0)),
            pl.BlockSpec((3 * H, 1), lambda i: (0, 0)),
        ],
        out_specs=pl.BlockSpec((H, BN), lambda i: (0, i)),
        out_shape=jax.ShapeDtypeStruct((H, _N_PAD), jnp.float32),
    )(partialsT, hT, root.T, conv_b[:, None], Wih.T, Whh.T, bih[:, None],
      bhh[:, None])


def _final_T_body(h_ref, xT_ref, extT_ref, wo1T_ref, bo1_ref, wo2T_ref,
                  bo2_ref, wc1T_ref, bc1_ref, wc2T_ref, bc2_ref, fg_ref,
                  pred_ref):
    BN = h_ref.shape[1]
    t = jnp.maximum(wo1T_ref[...] @ h_ref[...] + bo1_ref[...], 0.0)
    t = wo2T_ref[...] @ t + bo2_ref[...]
    atom = lax.broadcasted_iota(jnp.int32, (NUM_ATOMS, BN), 0)
    oh = jnp.where(atom == xT_ref[...], 1.0, 0.0)
    feat = jnp.concatenate([t, oh, extT_ref[...]], axis=0)
    nrm = jnp.sqrt(jnp.sum(feat * feat, axis=0, keepdims=True))
    fg = feat / jnp.maximum(nrm, 1e-12)
    fg_ref[...] = fg
    p = jnp.maximum(wc1T_ref[...] @ fg + bc1_ref[...], 0.0)
    pred_ref[...] = wc2T_ref[...] @ p + bc2_ref[...]


def _final_T(hT, xT_p, extT_p, Wo1, bo1, Wo2, bo2, Wc1, bc1, Wc2, bc2):
    FEAT = 64 + NUM_ATOMS + 3
    BN = 2048
    return pl.pallas_call(
        _final_T_body,
        grid=(_N_PAD // BN,),
        in_specs=[
            pl.BlockSpec((H, BN), lambda i: (0, i)),
            pl.BlockSpec((1, BN), lambda i: (0, i)),
            pl.BlockSpec((3, BN), lambda i: (0, i)),
            pl.BlockSpec((H, H), lambda i: (0, 0)),
            pl.BlockSpec((H, 1), lambda i: (0, 0)),
            pl.BlockSpec((64, H), lambda i: (0, 0)),
            pl.BlockSpec((64, 1), lambda i: (0, 0)),
            pl.BlockSpec((256, FEAT), lambda i: (0, 0)),
            pl.BlockSpec((256, 1), lambda i: (0, 0)),
            pl.BlockSpec((18, 256), lambda i: (0, 0)),
            pl.BlockSpec((18, 1), lambda i: (0, 0)),
        ],
        out_specs=[
            pl.BlockSpec((FEAT, BN), lambda i: (0, i)),
            pl.BlockSpec((18, BN), lambda i: (0, i)),
        ],
        out_shape=[
            jax.ShapeDtypeStruct((FEAT, _N_PAD), jnp.float32),
            jax.ShapeDtypeStruct((18, _N_PAD), jnp.float32),
        ],
    )(hT, xT_p, extT_p, Wo1.T, bo1[:, None], Wo2.T, bo2[:, None],
      Wc1.T, bc1[:, None], Wc2.T, bc2[:, None])


def kernel(x, edge_index, edge_attr, extended_feat, emb, eW1, eb1, eW2, eb2,
           root, conv_b, Wih, Whh, bih, bhh, Wo1, bo1, Wo2, bo2, Wc1, bc1,
           Wc2, bc2):
    E = edge_index.shape[1]
    src2 = jnp.pad(edge_index[0], (0, _E_PAD - E)).reshape(-1, _CHUNK)
    dst2 = jnp.pad(edge_index[1], (0, _E_PAD - E)).reshape(-1, _CHUNK)
    eaT_p = jnp.pad(edge_attr.T, ((0, 0), (0, _E_PAD - E)))
    xT_p = jnp.pad(x.T, ((0, 0), (0, _N_PAD - N_NODES)))
    extT_p = jnp.pad(extended_feat.T, ((0, 0), (0, _N_PAD - N_NODES)))
    extpT = jnp.pad(extended_feat.T,
                    ((H - 3, 0), (0, _N_PAD - N_NODES)))
    zeros = jnp.zeros((_N_PAD, H), jnp.float32)

    ewT = _edge_mlp_T(eaT_p, E, eW1, eb1, eW2, eb2)
    hT = _init_T(xT_p, extpT, emb)
    for _ in range(6):
        xgT = _sc_gather_T(hT, src2)
        msgT = _msg_T(xgT, ewT)
        partialsT = _sc_scatter_T(msgT, dst2, zeros)
        hT = _gru_T(partialsT, hT, root, conv_b, Wih, Whh, bih, bhh)
    fgT, predT = _final_T(hT, xT_p, extT_p, Wo1, bo1, Wo2, bo2, Wc1, bc1,
                          Wc2, bc2)
    fg = fgT[:, :N_NODES].T
    pred = predT[:, :N_NODES].T
    return (fg, pred)
